# Initial kernel scaffold; baseline (speedup 1.0000x reference)
import jax
import jax.numpy as jnp
from jax.experimental import pallas as pl


def kernel(keys, values, codebook):
    raise NotImplementedError("write your pallas kernel here")


# speedup vs baseline: 1.0329x; 1.0329x over previous
import functools

import jax
import jax.numpy as jnp
from jax import lax
from jax.experimental import pallas as pl
from jax.experimental.pallas import tpu as pltpu
from jax.experimental.pallas import tpu_sc as plsc

_H = 1024
_C = 256
_THRESH2 = 0.01
_BT = 512


def _prep_body(cb_ref, cbm_ref, cbt_ref, c2_ref):
    cb = cb_ref[...]
    c2 = jnp.sum(cb * cb, axis=1)
    mask = c2 > _THRESH2
    cbm_ref[...] = jnp.where(mask[:, None], cb, 0.0)
    cbt_ref[...] = cb.T
    c2_ref[...] = c2[None, :]


def _prep(codebook):
    return pl.pallas_call(
        _prep_body,
        out_shape=(
            jax.ShapeDtypeStruct((_C, _H), jnp.float32),
            jax.ShapeDtypeStruct((_H, _C), jnp.float32),
            jax.ShapeDtypeStruct((1, _C), jnp.float32),
        ),
    )(codebook)


def _argmin_body(c2_ref, cbt_ref, x_ref, idx_ref):
    x = x_ref[...]
    s = lax.dot_general(x, cbt_ref[...], (((1,), (0,)), ((), ())),
                        preferred_element_type=jnp.float32)
    d = c2_ref[...] - 2.0 * s
    m = jnp.min(d, axis=1, keepdims=True)
    iota = lax.broadcasted_iota(jnp.int32, d.shape, 1)
    idx = jnp.min(jnp.where(d <= m, iota, _C), axis=1)
    idx_ref[0, 0, :] = idx


def _argmin(c2, cbt, x2d):
    n = x2d.shape[0]
    nb = n // _BT
    out = pl.pallas_call(
        _argmin_body,
        grid=(nb,),
        in_specs=[
            pl.BlockSpec((1, _C), lambda i: (0, 0)),
            pl.BlockSpec((_H, _C), lambda i: (0, 0)),
            pl.BlockSpec((_BT, _H), lambda i: (i, 0)),
        ],
        out_specs=pl.BlockSpec((1, 1, _BT), lambda i: (i, 0, 0)),
        out_shape=jax.ShapeDtypeStruct((nb, 1, _BT), jnp.int32),
    )(c2, cbt, x2d)
    return out.reshape(n)


try:
    _INFO = plsc.get_sparse_core_info()
    _NC, _NS = _INFO.num_cores, _INFO.num_subcores
except Exception:
    _NC, _NS = 2, 16
_NW = _NC * _NS
_CH = 64


@functools.lru_cache(maxsize=None)
def _make_gather(n_rows):
    b_per_w = n_rows // _NW
    n_chunks = b_per_w // _CH
    mesh = plsc.VectorSubcoreMesh(core_axis_name="c", subcore_axis_name="s")

    @functools.partial(
        pl.kernel,
        mesh=mesh,
        out_type=jax.ShapeDtypeStruct((n_rows, _H), jnp.float32),
        scratch_types=[
            pltpu.VMEM((_CH,), jnp.int32),
            pltpu.VMEM((_CH, _H), jnp.float32),
            pltpu.SemaphoreType.DMA,
        ],
    )
    def gather_k(table_hbm, idx_hbm, out_hbm, idx_v, rows_v, sem):
        wid = lax.axis_index("s") * _NC + lax.axis_index("c")
        base = wid * b_per_w
        for k in range(n_chunks):
            off = base + k * _CH
            pltpu.sync_copy(idx_hbm.at[pl.ds(off, _CH)], idx_v)
            pltpu.async_copy(table_hbm.at[idx_v], rows_v, sem).wait()
            pltpu.sync_copy(rows_v, out_hbm.at[pl.ds(off, _CH)])

    return gather_k


def kernel(keys, values, codebook):
    b, s, h = keys.shape
    kd = keys.reshape(-1, h)
    vd = values.reshape(-1, h)
    cbm, cbt, c2 = _prep(codebook)
    idx_k = _argmin(c2, cbt, kd)
    idx_v = _argmin(c2, cbt, vd)
    gather = _make_gather(b * s)
    out_k = gather(cbm, idx_k)
    out_v = gather(cbm, idx_v)
    return out_k.reshape(b, s, h), out_v.reshape(b, s, h)
